# baseline (speedup 1.0000x reference)
import functools

import jax
import jax.numpy as jnp
from jax import lax
from jax.experimental import pallas as pl
from jax.experimental.pallas import tpu as pltpu
from jax.experimental.pallas import tpu_sc as plsc

_FEAT_DIM = 128
_BATCH = 4096
_LAMBDA_C = 1.0
_LANES = 16

_info = plsc.get_sparse_core_info()
_NC, _NS = _info.num_cores, _info.num_subcores
_NW = _NC * _NS
_BPW = _BATCH // _NW
_NCHUNK = 4
_RC = _BPW // _NCHUNK


def _center_loss_partials(feat, label, centers):
  mesh = plsc.VectorSubcoreMesh(core_axis_name="c", subcore_axis_name="s")

  @functools.partial(
      pl.kernel,
      mesh=mesh,
      out_type=jax.ShapeDtypeStruct((_NW, _LANES), jnp.float32),
      scratch_types=[
          pltpu.VMEM((_NCHUNK, _RC), jnp.int32),
          pltpu.VMEM((2, _RC, _FEAT_DIM), jnp.float32),
          pltpu.VMEM((2, _RC, _FEAT_DIM), jnp.float32),
          pltpu.VMEM((_LANES,), jnp.float32),
          pltpu.SemaphoreType.DMA,
          pltpu.SemaphoreType.DMA,
          pltpu.SemaphoreType.DMA,
          pltpu.SemaphoreType.DMA,
      ],
  )
  def k(feat_hbm, label_hbm, centers_hbm, out_hbm,
        idx_v, feat_v, rows_v, acc_v, gs0, gs1, fs0, fs1):
    wid = lax.axis_index("s") * _NC + lax.axis_index("c")
    pltpu.sync_copy(label_hbm.at[wid], idx_v)
    gsems = (gs0, gs1)
    fsems = (fs0, fs1)

    def fire(c):
      b = c % 2
      g = pltpu.async_copy(centers_hbm.at[idx_v.at[c]], rows_v.at[b], gsems[b])
      f = pltpu.async_copy(feat_hbm.at[wid, c], feat_v.at[b], fsems[b])
      return g, f

    inflight = fire(0)
    acc = jnp.zeros((_LANES,), jnp.float32)
    for c in range(_NCHUNK):
      b = c % 2
      g, f = inflight
      g.wait()
      f.wait()
      if c + 1 < _NCHUNK:
        inflight = fire(c + 1)

      def body(r, a, b=b):
        for d in range(_FEAT_DIM // _LANES):
          x = feat_v[b, r, pl.ds(d * _LANES, _LANES)]
          y = rows_v[b, r, pl.ds(d * _LANES, _LANES)]
          diff = x - y
          a = a + diff * diff
        return a

      acc = lax.fori_loop(0, _RC, body, acc)

    acc_v[...] = acc
    pltpu.sync_copy(acc_v, out_hbm.at[wid])

  return k(feat, label, centers)


def kernel(feat, label, centers):
  label = label.astype(jnp.int32).reshape(_NW, _NCHUNK, _RC)
  feat_r = feat.reshape(_NW, _NCHUNK, _RC, _FEAT_DIM)
  partials = _center_loss_partials(feat_r, label, centers)
  return _LAMBDA_C / 2.0 / _BATCH * jnp.sqrt(jnp.sum(partials))


# speedup vs baseline: 1.0439x; 1.0439x over previous
import functools

import jax
import jax.numpy as jnp
from jax import lax
from jax.experimental import pallas as pl
from jax.experimental.pallas import tpu as pltpu
from jax.experimental.pallas import tpu_sc as plsc

_FEAT_DIM = 128
_BATCH = 4096
_LAMBDA_C = 1.0
_LANES = 16

_info = plsc.get_sparse_core_info()
_NC, _NS = _info.num_cores, _info.num_subcores
_NW = _NC * _NS
_BPW = _BATCH // _NW
_NCHUNK = 2
_RC = _BPW // _NCHUNK


def _center_loss_partials(feat, label, centers):
  mesh = plsc.VectorSubcoreMesh(core_axis_name="c", subcore_axis_name="s")

  @functools.partial(
      pl.kernel,
      mesh=mesh,
      out_type=jax.ShapeDtypeStruct((_NW, _LANES), jnp.float32),
      scratch_types=[
          pltpu.VMEM((_NCHUNK, _RC), jnp.int32),
          pltpu.VMEM((_BPW, _FEAT_DIM), jnp.float32),
          pltpu.VMEM((_NCHUNK, _RC, _FEAT_DIM), jnp.float32),
          pltpu.VMEM((_LANES,), jnp.float32),
          pltpu.SemaphoreType.DMA,
          pltpu.SemaphoreType.DMA,
          pltpu.SemaphoreType.DMA,
      ],
  )
  def k(feat_hbm, label_hbm, centers_hbm, out_hbm,
        idx_v, feat_v, rows_v, acc_v, fsem, gs0, gs1):
    wid = lax.axis_index("s") * _NC + lax.axis_index("c")
    fcopy = pltpu.async_copy(feat_hbm.at[wid], feat_v, fsem)
    pltpu.sync_copy(label_hbm.at[wid], idx_v)
    gsems = (gs0, gs1)
    gathers = [
        pltpu.async_copy(centers_hbm.at[idx_v.at[c]], rows_v.at[c], gsems[c])
        for c in range(_NCHUNK)
    ]
    fcopy.wait()

    acc = jnp.zeros((_LANES,), jnp.float32)
    for c in range(_NCHUNK):
      gathers[c].wait()

      def body(r, a, c=c):
        for d in range(_FEAT_DIM // _LANES):
          x = feat_v[c * _RC + r, pl.ds(d * _LANES, _LANES)]
          y = rows_v[c, r, pl.ds(d * _LANES, _LANES)]
          diff = x - y
          a = a + diff * diff
        return a

      acc = lax.fori_loop(0, _RC, body, acc)

    acc_v[...] = acc
    pltpu.sync_copy(acc_v, out_hbm.at[wid])

  return k(feat, label, centers)


def kernel(feat, label, centers):
  label = label.astype(jnp.int32).reshape(_NW, _NCHUNK, _RC)
  feat_r = feat.reshape(_NW, _BPW, _FEAT_DIM)
  partials = _center_loss_partials(feat_r, label, centers)
  return _LAMBDA_C / 2.0 / _BATCH * jnp.sqrt(jnp.sum(partials))
